# baseline (speedup 1.0000x reference)
import functools

import jax
import jax.numpy as jnp
from jax import lax
from jax.experimental import pallas as pl
from jax.experimental.pallas import tpu as pltpu
from jax.experimental.pallas import tpu_sc as plsc

_B, _H, _W, _C = 2, 512, 512, 96
_P = _B * _H * _W
_NC, _NS, _L = 2, 16, 16
_NW = _NC * _NS
_CHUNK = 64
_PER_W = _P // _NW
_NCH = _PER_W // _CHUNK
_NG = _CHUNK // _L
_NSET = 2


def _warp_body(img_hbm, flo_hbm, out_hbm, *bufs):
  sets = (bufs[:9], bufs[9:18])
  fsets = (bufs[18:23], bufs[23:28])
  wid = lax.axis_index("s") * _NC + lax.axis_index("c")
  lane = lax.iota(jnp.int32, _L)

  def fire_flow(ch, fset):
    (fxi_v, fyi_v, fx_v, fy_v, semf) = fset
    base = wid * _PER_W + ch * _CHUNK
    for v in range(_NG):
      s = pl.ds(v * _L, _L)
      ixb = 2 * (base + v * _L) + 2 * lane
      fxi_v[s] = ixb
      fyi_v[s] = ixb + 1
    pltpu.async_copy(flo_hbm.at[fxi_v], fx_v, semf)
    pltpu.async_copy(flo_hbm.at[fyi_v], fy_v, semf)

  def prep(ch, bufset, fset):
    (ia_v, ib_v, ic_v, id_v, wvs_v, ro_v, o_v, semg, semw) = bufset
    (fxi_v, fyi_v, fx_v, fy_v, semf) = fset
    base = wid * _PER_W + ch * _CHUNK
    pltpu.make_async_copy(flo_hbm.at[fxi_v], fx_v, semf).wait()
    pltpu.make_async_copy(flo_hbm.at[fyi_v], fy_v, semf).wait()
    jbase = base & (_W - 1)
    irow_f = ((base >> 9) & (_H - 1)).astype(jnp.float32)
    boff = (base >> 18) << 18

    for v in range(_NG):
      s = pl.ds(v * _L, _L)
      x = (jbase + v * _L + lane).astype(jnp.float32) + fx_v[s]
      y = irow_f + fy_v[s]
      xt = x.astype(jnp.int32)
      yt = y.astype(jnp.int32)
      x0 = jnp.clip(xt, 0, _W - 1)
      x1 = jnp.clip(xt + 1, 0, _W - 1)
      y0 = jnp.clip(yt, 0, _H - 1)
      y1 = jnp.clip(yt + 1, 0, _H - 1)
      x0f = x0.astype(jnp.float32)
      x1f = x1.astype(jnp.float32)
      y0f = y0.astype(jnp.float32)
      y1f = y1.astype(jnp.float32)
      ia_v[s] = boff + y0 * _W + x0
      ib_v[s] = boff + y1 * _W + x0
      ic_v[s] = boff + y0 * _W + x1
      id_v[s] = boff + y1 * _W + x1
      wvs_v[0, s] = (x1f - x) * (y1f - y)
      wvs_v[1, s] = (x1f - x) * (y - y0f)
      wvs_v[2, s] = (x - x0f) * (y1f - y)
      wvs_v[3, s] = (x - x0f) * (y - y0f)

    for q, idx_v in enumerate((ia_v, ib_v, ic_v, id_v)):
      pltpu.async_copy(img_hbm.at[idx_v], ro_v.at[q], semg)

  def finish(ch, bufset):
    (ia_v, ib_v, ic_v, id_v, wvs_v, ro_v, o_v, semg, semw) = bufset
    base = wid * _PER_W + ch * _CHUNK
    for q, idx_v in enumerate((ia_v, ib_v, ic_v, id_v)):
      pltpu.make_async_copy(img_hbm.at[idx_v], ro_v.at[q], semg).wait()

    @pl.when(ch >= _NSET)
    def _wait_prev_write():
      pltpu.make_async_copy(o_v, out_hbm.at[pl.ds(base, _CHUNK)], semw).wait()

    def grp_body(g, c2):
      gs = pl.ds(pl.multiple_of(g * _L, _L), _L)
      wag = wvs_v[0, gs]
      wbg = wvs_v[1, gs]
      wcg = wvs_v[2, gs]
      wdg = wvs_v[3, gs]
      for k in range(_L):
        p = g * _L + k
        wa = wag[k]
        wb = wbg[k]
        wc = wcg[k]
        wd = wdg[k]
        for cg in range(_C // _L):
          cs = pl.ds(cg * _L, _L)
          o_v[p, cs] = (ro_v[0, p, cs] * wa + ro_v[1, p, cs] * wb
                        + ro_v[2, p, cs] * wc + ro_v[3, p, cs] * wd)
      return c2

    lax.fori_loop(0, _NG, grp_body, 0)
    pltpu.async_copy(o_v, out_hbm.at[pl.ds(base, _CHUNK)], semw)

  fire_flow(0, fsets[0])
  fire_flow(1, fsets[1])
  prep(0, sets[0], fsets[0])
  fire_flow(2, fsets[0])
  prep(1, sets[1], fsets[1])
  fire_flow(3, fsets[1])

  def round_body(g, carry):
    for s in range(_NSET):
      c = g * _NSET + s
      finish(c, sets[s])
      prep(c + _NSET, sets[s], fsets[s])

      @pl.when(c + 2 * _NSET < _NCH)
      def _prefetch_flow(c=c, s=s):
        fire_flow(c + 2 * _NSET, fsets[s])

    return carry

  lax.fori_loop(0, _NCH // _NSET - 1, round_body, 0)

  for s in range(_NSET):
    finish(_NCH - _NSET + s, sets[s])

  for s in range(_NSET):
    ch = _NCH - _NSET + s
    baseS = wid * _PER_W + ch * _CHUNK
    pltpu.make_async_copy(sets[s][6], out_hbm.at[pl.ds(baseS, _CHUNK)],
                          sets[s][8]).wait()


def _buf_set():
  return [
      pltpu.VMEM((_CHUNK,), jnp.int32),
      pltpu.VMEM((_CHUNK,), jnp.int32),
      pltpu.VMEM((_CHUNK,), jnp.int32),
      pltpu.VMEM((_CHUNK,), jnp.int32),
      pltpu.VMEM((4, _CHUNK), jnp.float32),
      pltpu.VMEM((4, _CHUNK, _C), jnp.float32),
      pltpu.VMEM((_CHUNK, _C), jnp.float32),
      pltpu.SemaphoreType.DMA,
      pltpu.SemaphoreType.DMA,
  ]


def _flow_set():
  return [
      pltpu.VMEM((_CHUNK,), jnp.int32),
      pltpu.VMEM((_CHUNK,), jnp.int32),
      pltpu.VMEM((_CHUNK,), jnp.float32),
      pltpu.VMEM((_CHUNK,), jnp.float32),
      pltpu.SemaphoreType.DMA,
  ]


@functools.lru_cache(maxsize=None)
def _build():
  mesh = plsc.VectorSubcoreMesh(core_axis_name="c", subcore_axis_name="s",
                                num_cores=_NC, num_subcores=_NS)
  return pl.kernel(
      _warp_body,
      out_type=jax.ShapeDtypeStruct((_P, _C), jnp.float32),
      mesh=mesh,
      compiler_params=pltpu.CompilerParams(use_tc_tiling_on_sc=False),
      scratch_types=_buf_set() * _NSET + _flow_set() * _NSET,
  )


def kernel(img, flo):
  B, H, W, C = img.shape
  out = _build()(img.reshape(B * H * W, C), flo.reshape(-1))
  return out.reshape(B, H, W, C)


# speedup vs baseline: 1.0762x; 1.0762x over previous
import functools

import jax
import jax.numpy as jnp
from jax import lax
from jax.experimental import pallas as pl
from jax.experimental.pallas import tpu as pltpu
from jax.experimental.pallas import tpu_sc as plsc

_B, _H, _W, _C = 2, 512, 512, 96
_P = _B * _H * _W
_NC, _NS, _L = 2, 16, 16
_NW = _NC * _NS
_CHUNK = 32
_PER_W = _P // _NW
_NCH = _PER_W // _CHUNK
_NG = _CHUNK // _L
_NSET = 4


def _warp_body(img_hbm, flox_hbm, floy_hbm, out_hbm, *bufs):
  sets = tuple(bufs[9 * i:9 * (i + 1)] for i in range(_NSET))
  fsets = tuple(bufs[9 * _NSET + 3 * i:9 * _NSET + 3 * (i + 1)]
                for i in range(_NSET))
  wid = lax.axis_index("s") * _NC + lax.axis_index("c")
  lane = lax.iota(jnp.int32, _L)

  def fire_flow(ch, fset):
    (fx_v, fy_v, semf) = fset
    base = wid * _PER_W + ch * _CHUNK
    pltpu.async_copy(flox_hbm.at[pl.ds(base, _CHUNK)], fx_v, semf)
    pltpu.async_copy(floy_hbm.at[pl.ds(base, _CHUNK)], fy_v, semf)

  def prep(ch, bufset, fset):
    (ia_v, ib_v, ic_v, id_v, wvs_v, ro_v, o_v, semg, semw) = bufset
    (fx_v, fy_v, semf) = fset
    base = wid * _PER_W + ch * _CHUNK
    pltpu.make_async_copy(flox_hbm.at[pl.ds(base, _CHUNK)], fx_v, semf).wait()
    pltpu.make_async_copy(floy_hbm.at[pl.ds(base, _CHUNK)], fy_v, semf).wait()
    jbase = base & (_W - 1)
    irow_f = ((base >> 9) & (_H - 1)).astype(jnp.float32)
    boff = (base >> 18) << 18

    for v in range(_NG):
      s = pl.ds(v * _L, _L)
      x = (jbase + v * _L + lane).astype(jnp.float32) + fx_v[s]
      y = irow_f + fy_v[s]
      xt = x.astype(jnp.int32)
      yt = y.astype(jnp.int32)
      x0 = jnp.clip(xt, 0, _W - 1)
      x1 = jnp.clip(xt + 1, 0, _W - 1)
      y0 = jnp.clip(yt, 0, _H - 1)
      y1 = jnp.clip(yt + 1, 0, _H - 1)
      x0f = x0.astype(jnp.float32)
      x1f = x1.astype(jnp.float32)
      y0f = y0.astype(jnp.float32)
      y1f = y1.astype(jnp.float32)
      ia_v[s] = boff + y0 * _W + x0
      ib_v[s] = boff + y1 * _W + x0
      ic_v[s] = boff + y0 * _W + x1
      id_v[s] = boff + y1 * _W + x1
      wvs_v[0, s] = (x1f - x) * (y1f - y)
      wvs_v[1, s] = (x1f - x) * (y - y0f)
      wvs_v[2, s] = (x - x0f) * (y1f - y)
      wvs_v[3, s] = (x - x0f) * (y - y0f)

    for q, idx_v in enumerate((ia_v, ib_v, ic_v, id_v)):
      pltpu.async_copy(img_hbm.at[idx_v], ro_v.at[q], semg)

  def finish(ch, bufset):
    (ia_v, ib_v, ic_v, id_v, wvs_v, ro_v, o_v, semg, semw) = bufset
    base = wid * _PER_W + ch * _CHUNK
    for q, idx_v in enumerate((ia_v, ib_v, ic_v, id_v)):
      pltpu.make_async_copy(img_hbm.at[idx_v], ro_v.at[q], semg).wait()

    @pl.when(ch >= _NSET)
    def _wait_prev_write():
      pltpu.make_async_copy(o_v, out_hbm.at[pl.ds(base, _CHUNK)], semw).wait()

    def grp_body(g, c2):
      gs = pl.ds(pl.multiple_of(g * _L, _L), _L)
      wag = wvs_v[0, gs]
      wbg = wvs_v[1, gs]
      wcg = wvs_v[2, gs]
      wdg = wvs_v[3, gs]
      for k in range(_L):
        p = g * _L + k
        wa = wag[k]
        wb = wbg[k]
        wc = wcg[k]
        wd = wdg[k]
        for cg in range(_C // _L):
          cs = pl.ds(cg * _L, _L)
          o_v[p, cs] = (ro_v[0, p, cs] * wa + ro_v[1, p, cs] * wb
                        + ro_v[2, p, cs] * wc + ro_v[3, p, cs] * wd)
      return c2

    lax.fori_loop(0, _NG, grp_body, 0)
    pltpu.async_copy(o_v, out_hbm.at[pl.ds(base, _CHUNK)], semw)

  for s in range(_NSET):
    fire_flow(s, fsets[s])
  for s in range(_NSET):
    prep(s, sets[s], fsets[s])
    fire_flow(s + _NSET, fsets[s])

  def round_body(g, carry):
    for s in range(_NSET):
      c = g * _NSET + s
      finish(c, sets[s])
      prep(c + _NSET, sets[s], fsets[s])

      @pl.when(c + 2 * _NSET < _NCH)
      def _prefetch_flow(c=c, s=s):
        fire_flow(c + 2 * _NSET, fsets[s])

    return carry

  lax.fori_loop(0, _NCH // _NSET - 1, round_body, 0)

  for s in range(_NSET):
    finish(_NCH - _NSET + s, sets[s])

  for s in range(_NSET):
    ch = _NCH - _NSET + s
    baseS = wid * _PER_W + ch * _CHUNK
    pltpu.make_async_copy(sets[s][6], out_hbm.at[pl.ds(baseS, _CHUNK)],
                          sets[s][8]).wait()


def _buf_set():
  return [
      pltpu.VMEM((_CHUNK,), jnp.int32),
      pltpu.VMEM((_CHUNK,), jnp.int32),
      pltpu.VMEM((_CHUNK,), jnp.int32),
      pltpu.VMEM((_CHUNK,), jnp.int32),
      pltpu.VMEM((4, _CHUNK), jnp.float32),
      pltpu.VMEM((4, _CHUNK, _C), jnp.float32),
      pltpu.VMEM((_CHUNK, _C), jnp.float32),
      pltpu.SemaphoreType.DMA,
      pltpu.SemaphoreType.DMA,
  ]


def _flow_set():
  return [
      pltpu.VMEM((_CHUNK,), jnp.float32),
      pltpu.VMEM((_CHUNK,), jnp.float32),
      pltpu.SemaphoreType.DMA,
  ]


@functools.lru_cache(maxsize=None)
def _build():
  mesh = plsc.VectorSubcoreMesh(core_axis_name="c", subcore_axis_name="s",
                                num_cores=_NC, num_subcores=_NS)
  return pl.kernel(
      _warp_body,
      out_type=jax.ShapeDtypeStruct((_P, _C), jnp.float32),
      mesh=mesh,
      compiler_params=pltpu.CompilerParams(use_tc_tiling_on_sc=False),
      scratch_types=_buf_set() * _NSET + _flow_set() * _NSET,
  )


def kernel(img, flo):
  B, H, W, C = img.shape
  flo2 = flo.reshape(_P, 2)
  out = _build()(img.reshape(B * H * W, C),
                 flo2[:, 0], flo2[:, 1])
  return out.reshape(B, H, W, C)


# speedup vs baseline: 1.3035x; 1.2112x over previous
import functools

import jax
import jax.numpy as jnp
from jax import lax
from jax.experimental import pallas as pl
from jax.experimental.pallas import tpu as pltpu
from jax.experimental.pallas import tpu_sc as plsc

_B, _H, _W, _C = 2, 512, 512, 96
_P = _B * _H * _W
_NC, _NS, _L = 2, 16, 16
_NW = _NC * _NS
_CHUNK = 64
_PER_W = _P // _NW
_NCH = _PER_W // _CHUNK
_NG = _CHUNK // _L
_NSET = 2


def _warp_body(img_hbm, flox_hbm, floy_hbm, out_hbm, *bufs):
  sets = tuple(bufs[9 * i:9 * (i + 1)] for i in range(_NSET))
  fsets = tuple(bufs[9 * _NSET + 3 * i:9 * _NSET + 3 * (i + 1)]
                for i in range(_NSET))
  wid = lax.axis_index("s") * _NC + lax.axis_index("c")
  lane = lax.iota(jnp.int32, _L)

  def fire_flow(ch, fset):
    (fx_v, fy_v, semf) = fset
    base = wid * _PER_W + ch * _CHUNK
    pltpu.async_copy(flox_hbm.at[pl.ds(base, _CHUNK)], fx_v, semf)
    pltpu.async_copy(floy_hbm.at[pl.ds(base, _CHUNK)], fy_v, semf)

  def prep(ch, bufset, fset):
    (ia_v, ib_v, ic_v, id_v, wvs_v, ro_v, o_v, semg, semw) = bufset
    (fx_v, fy_v, semf) = fset
    base = wid * _PER_W + ch * _CHUNK
    pltpu.make_async_copy(flox_hbm.at[pl.ds(base, _CHUNK)], fx_v, semf).wait()
    pltpu.make_async_copy(floy_hbm.at[pl.ds(base, _CHUNK)], fy_v, semf).wait()
    jbase = base & (_W - 1)
    irow_f = ((base >> 9) & (_H - 1)).astype(jnp.float32)
    boff = (base >> 18) << 18

    for v in range(_NG):
      s = pl.ds(v * _L, _L)
      x = (jbase + v * _L + lane).astype(jnp.float32) + fx_v[s]
      y = irow_f + fy_v[s]
      xt = x.astype(jnp.int32)
      yt = y.astype(jnp.int32)
      x0 = jnp.clip(xt, 0, _W - 1)
      x1 = jnp.clip(xt + 1, 0, _W - 1)
      y0 = jnp.clip(yt, 0, _H - 1)
      y1 = jnp.clip(yt + 1, 0, _H - 1)
      x0f = x0.astype(jnp.float32)
      x1f = x1.astype(jnp.float32)
      y0f = y0.astype(jnp.float32)
      y1f = y1.astype(jnp.float32)
      ia_v[s] = boff + y0 * _W + x0
      ib_v[s] = boff + y1 * _W + x0
      ic_v[s] = boff + y0 * _W + x1
      id_v[s] = boff + y1 * _W + x1
      wvs_v[0, s] = (x1f - x) * (y1f - y)
      wvs_v[1, s] = (x1f - x) * (y - y0f)
      wvs_v[2, s] = (x - x0f) * (y1f - y)
      wvs_v[3, s] = (x - x0f) * (y - y0f)

    for q, idx_v in enumerate((ia_v, ib_v, ic_v, id_v)):
      pltpu.async_copy(img_hbm.at[idx_v], ro_v.at[q], semg)

  def finish(ch, bufset):
    (ia_v, ib_v, ic_v, id_v, wvs_v, ro_v, o_v, semg, semw) = bufset
    base = wid * _PER_W + ch * _CHUNK
    for q, idx_v in enumerate((ia_v, ib_v, ic_v, id_v)):
      pltpu.make_async_copy(img_hbm.at[idx_v], ro_v.at[q], semg).wait()

    @pl.when(ch >= _NSET)
    def _wait_prev_write():
      pltpu.make_async_copy(o_v, out_hbm.at[pl.ds(base, _CHUNK)], semw).wait()

    def grp_body(g, c2):
      gs = pl.ds(pl.multiple_of(g * _L, _L), _L)
      wag = wvs_v[0, gs]
      wbg = wvs_v[1, gs]
      wcg = wvs_v[2, gs]
      wdg = wvs_v[3, gs]
      for k in range(_L):
        p = g * _L + k
        wa = wag[k]
        wb = wbg[k]
        wc = wcg[k]
        wd = wdg[k]
        for cg in range(_C // _L):
          cs = pl.ds(cg * _L, _L)
          o_v[p, cs] = (ro_v[0, p, cs] * wa + ro_v[1, p, cs] * wb
                        + ro_v[2, p, cs] * wc + ro_v[3, p, cs] * wd)
      return c2

    lax.fori_loop(0, _NG, grp_body, 0)
    pltpu.async_copy(o_v, out_hbm.at[pl.ds(base, _CHUNK)], semw)

  for s in range(_NSET):
    fire_flow(s, fsets[s])
  for s in range(_NSET):
    prep(s, sets[s], fsets[s])
    fire_flow(s + _NSET, fsets[s])

  def round_body(g, carry):
    for s in range(_NSET):
      c = g * _NSET + s
      finish(c, sets[s])
      prep(c + _NSET, sets[s], fsets[s])

      @pl.when(c + 2 * _NSET < _NCH)
      def _prefetch_flow(c=c, s=s):
        fire_flow(c + 2 * _NSET, fsets[s])

    return carry

  lax.fori_loop(0, _NCH // _NSET - 1, round_body, 0)

  for s in range(_NSET):
    finish(_NCH - _NSET + s, sets[s])

  for s in range(_NSET):
    ch = _NCH - _NSET + s
    baseS = wid * _PER_W + ch * _CHUNK
    pltpu.make_async_copy(sets[s][6], out_hbm.at[pl.ds(baseS, _CHUNK)],
                          sets[s][8]).wait()


def _buf_set():
  return [
      pltpu.VMEM((_CHUNK,), jnp.int32),
      pltpu.VMEM((_CHUNK,), jnp.int32),
      pltpu.VMEM((_CHUNK,), jnp.int32),
      pltpu.VMEM((_CHUNK,), jnp.int32),
      pltpu.VMEM((4, _CHUNK), jnp.float32),
      pltpu.VMEM((4, _CHUNK, _C), jnp.float32),
      pltpu.VMEM((_CHUNK, _C), jnp.float32),
      pltpu.SemaphoreType.DMA,
      pltpu.SemaphoreType.DMA,
  ]


def _flow_set():
  return [
      pltpu.VMEM((_CHUNK,), jnp.float32),
      pltpu.VMEM((_CHUNK,), jnp.float32),
      pltpu.SemaphoreType.DMA,
  ]


@functools.lru_cache(maxsize=None)
def _build():
  mesh = plsc.VectorSubcoreMesh(core_axis_name="c", subcore_axis_name="s",
                                num_cores=_NC, num_subcores=_NS)
  return pl.kernel(
      _warp_body,
      out_type=jax.ShapeDtypeStruct((_P, _C), jnp.float32),
      mesh=mesh,
      compiler_params=pltpu.CompilerParams(use_tc_tiling_on_sc=False),
      scratch_types=_buf_set() * _NSET + _flow_set() * _NSET,
  )


def kernel(img, flo):
  B, H, W, C = img.shape
  flo2 = flo.reshape(_P, 2)
  out = _build()(img.reshape(B * H * W, C),
                 flo2[:, 0], flo2[:, 1])
  return out.reshape(B, H, W, C)
